# Initial kernel scaffold; baseline (speedup 1.0000x reference)
import jax
import jax.numpy as jnp
from jax.experimental import pallas as pl


def kernel(pred_logits, pred_obj, pred_boxes, pred_unk, target_sizes):
    raise NotImplementedError("write your pallas kernel here")


# speedup vs baseline: 9.7818x; 9.7818x over previous
import functools

import jax
import jax.numpy as jnp
from jax.experimental import pallas as pl

_INTERPRET = False

_TEMP = 1.3
_BETA = 1.5
_K_OUT = 100
_K_CAND = 128
_N_VALID = 81


def _row_max_kernel(logits_ref, obj_ref, unk_ref, m_ref):
    x = logits_ref[...]
    cmask = jax.lax.broadcasted_iota(jnp.int32, x.shape, 3) < _N_VALID
    lmax = jnp.max(jnp.where(cmask, x, -1e30), axis=-1)
    obj = jnp.exp(-_TEMP * obj_ref[...])
    u = jax.nn.sigmoid(unk_ref[...])
    s = jax.nn.sigmoid(lmax)
    maxk = jnp.where(s > 0.2, s, 0.0)
    w = 1.0 - _BETA * u
    kpk = jnp.where((u > maxk) | (w < 0.0), 0.0, (obj * maxk) * w)
    punk = (obj * u) * (1.0 - maxk)
    m_ref[...] = jnp.maximum(punk, kpk)


def _row_max(pred_logits, pred_obj, pred_unk):
    B, N, C = pred_logits.shape
    rows = B * N
    G, SR = 50, 25
    logits4 = pred_logits.reshape(G, SR, 128, C)
    obj3 = pred_obj.reshape(G, SR, 128)
    unk3 = pred_unk.reshape(G, SR, 128)
    m = pl.pallas_call(
        _row_max_kernel,
        grid=(G,),
        in_specs=[
            pl.BlockSpec((1, SR, 128, C), lambda i: (i, 0, 0, 0)),
            pl.BlockSpec((1, SR, 128), lambda i: (i, 0, 0)),
            pl.BlockSpec((1, SR, 128), lambda i: (i, 0, 0)),
        ],
        out_specs=pl.BlockSpec((1, SR, 128), lambda i: (i, 0, 0)),
        out_shape=jax.ShapeDtypeStruct((G, SR, 128), jnp.float32),
        interpret=_INTERPRET,
    )(logits4, obj3, unk3)
    return m.reshape(B, N)


def kernel(pred_logits, pred_obj, pred_boxes, pred_unk, target_sizes):
    B, N, C = pred_logits.shape
    M = _row_max(pred_logits, pred_obj, pred_unk)

    _, cand = jax.lax.top_k(M, _K_CAND)
    cand = jnp.sort(cand, axis=-1)
    lg = jnp.take_along_axis(pred_logits, cand[:, :, None], axis=1)
    obj_k = jnp.exp(-_TEMP * jnp.take_along_axis(pred_obj, cand, axis=1))
    u_k = jax.nn.sigmoid(jnp.take_along_axis(pred_unk, cand, axis=1))
    kp = jax.nn.sigmoid(lg)
    kp = kp * (kp > 0.2).astype(jnp.float32)
    cidx = jnp.arange(C)
    kp = jnp.where((cidx >= _N_VALID)[None, None, :], 0.0, kp)
    maxk_k = jnp.max(kp[:, :, : C - 1], axis=-1)
    pk = obj_k[..., None] * kp[:, :, : C - 1] * (1.0 - _BETA * u_k[..., None])
    pk = jnp.where((u_k > maxk_k)[..., None], 0.0, pk)
    pu = obj_k * u_k * (1.0 - maxk_k)
    ent = jnp.concatenate([pk, pu[..., None]], axis=-1)
    vals, e = jax.lax.top_k(ent.reshape(B, _K_CAND * C), _K_OUT)
    j = e // C
    labels = e % C
    n_sel = jnp.take_along_axis(cand, j, axis=1)
    bx = jnp.take_along_axis(pred_boxes, n_sel[:, :, None], axis=1)
    cx, cy, wd, ht = bx[..., 0], bx[..., 1], bx[..., 2], bx[..., 3]
    bx = jnp.stack(
        [cx - 0.5 * wd, cy - 0.5 * ht, cx + 0.5 * wd, cy + 0.5 * ht], axis=-1
    )
    img_h = target_sizes[:, 0].astype(jnp.float32)
    img_w = target_sizes[:, 1].astype(jnp.float32)
    scale = jnp.stack([img_w, img_h, img_w, img_h], axis=1)
    bx = bx * scale[:, None, :]
    return vals, labels, bx


# speedup vs baseline: 11.0943x; 1.1342x over previous
import jax
import jax.numpy as jnp
from jax import lax
from jax.experimental import pallas as pl

_INTERPRET = False

_TEMP = 1.3
_BETA = 1.5
_K_OUT = 100
_K = 128
_NV = 81


def _dot(a, b):
    return lax.dot_general(a, b, (((1,), (0,)), ((), ())),
                           precision=lax.Precision.HIGHEST,
                           preferred_element_type=jnp.float32)


def _row_max_kernel(logits_ref, obj_ref, unk_ref, m_ref):
    x = logits_ref[...]
    cmask = lax.broadcasted_iota(jnp.int32, x.shape, 3) < _NV
    lmax = jnp.max(jnp.where(cmask, x, -1e30), axis=-1)
    obj = jnp.exp(-_TEMP * obj_ref[...])
    u = jax.nn.sigmoid(unk_ref[...])
    s = jax.nn.sigmoid(lmax)
    maxk = jnp.where(s > 0.2, s, 0.0)
    w = 1.0 - _BETA * u
    kpk = jnp.where((u > maxk) | (w < 0.0), 0.0, (obj * maxk) * w)
    punk = (obj * u) * (1.0 - maxk)
    m_ref[...] = jnp.maximum(punk, kpk)


def _row_max(pred_logits, pred_obj, pred_unk):
    B, N, C = pred_logits.shape
    G, SR = 50, 25
    m = pl.pallas_call(
        _row_max_kernel,
        grid=(G,),
        in_specs=[
            pl.BlockSpec((1, SR, 128, C), lambda i: (i, 0, 0, 0)),
            pl.BlockSpec((1, SR, 128), lambda i: (i, 0, 0)),
            pl.BlockSpec((1, SR, 128), lambda i: (i, 0, 0)),
        ],
        out_specs=pl.BlockSpec((1, SR, 128), lambda i: (i, 0, 0)),
        out_shape=jax.ShapeDtypeStruct((G, SR, 128), jnp.float32),
        interpret=_INTERPRET,
    )(
        pred_logits.reshape(G, SR, 128, C),
        pred_obj.reshape(G, SR, 128),
        pred_unk.reshape(G, SR, 128),
    )
    return m.reshape(B, N)


def _thr_kernel(m_ref, thr_ref):
    bits = lax.bitcast_convert_type(m_ref[...], jnp.int32)
    B = bits.shape[0]

    def body(_, lohi):
        lo, hi = lohi
        mid = lo + ((hi - lo + 1) >> 1)
        cnt = jnp.sum((bits >= mid).astype(jnp.int32), axis=1, keepdims=True)
        ge = cnt >= _K
        return jnp.where(ge, mid, lo), jnp.where(ge, hi, mid - 1)

    lo0 = jnp.zeros((B, 1), jnp.int32)
    hi0 = jnp.full((B, 1), 0x7F800000, jnp.int32)
    lo, _ = lax.fori_loop(0, 31, body, (lo0, hi0))
    thr_ref[...] = jnp.broadcast_to(lo[:, :, None], thr_ref.shape)


def _thresholds(M):
    B, N = M.shape
    return pl.pallas_call(
        _thr_kernel,
        grid=(1,),
        in_specs=[pl.BlockSpec((B, N), lambda i: (0, 0))],
        out_specs=pl.BlockSpec((B, 8, 128), lambda i: (0, 0, 0)),
        out_shape=jax.ShapeDtypeStruct((B, 8, 128), jnp.int32),
        interpret=_INTERPRET,
    )(M)


def _select_kernel(logits_ref, small_ref, m_ref, thr_ref, tsz_ref,
                   sc_ref, lb_ref, bx_ref):
    N, C = logits_ref.shape[1], logits_ref.shape[2]
    K = _K
    f32, i32 = jnp.float32, jnp.int32

    mbits = lax.bitcast_convert_type(m_ref[0], i32)
    t1 = jnp.max(thr_ref[0])
    qual1 = (mbits >= t1).astype(i32)
    cum = qual1
    sh = 1
    while sh < N:
        cum = cum + jnp.concatenate(
            [jnp.zeros((1, sh), i32), cum[:, : N - sh]], axis=1)
        sh *= 2
    cnt1 = jnp.max(cum, axis=1, keepdims=True)
    ir = lax.broadcasted_iota(i32, (K, K), 0)
    ic = lax.broadcasted_iota(i32, (K, K), 1)
    eye = (ic == ir).astype(f32)
    st = small_ref[0]
    CH = 2500
    LG = jnp.zeros((K, C), f32)
    SMt = jnp.zeros((8, K), f32)
    for t in range(N // CH):
        cums = cum[:, t * CH:(t + 1) * CH]
        quals = qual1[:, t * CH:(t + 1) * CH]
        jr = lax.broadcasted_iota(i32, (K, CH), 0)
        Oc = ((cums == jr + 1) & (quals > 0)).astype(f32)
        LG = LG + _dot(Oc, logits_ref[0, t * CH:(t + 1) * CH, :])
        SMt = SMt + lax.dot_general(
            st[:, t * CH:(t + 1) * CH], Oc, (((1,), (1,)), ((), ())),
            precision=lax.Precision.HIGHEST,
            preferred_element_type=f32)
    SM = lax.dot_general(eye, SMt, (((1,), (1,)), ((), ())),
                         precision=lax.Precision.HIGHEST,
                         preferred_element_type=f32)
    valid = lax.broadcasted_iota(i32, (K, 1), 0) < cnt1

    obj = jnp.exp(-_TEMP * SM[:, 0:1])
    u = jax.nn.sigmoid(SM[:, 1:2])
    kp = jax.nn.sigmoid(LG)
    kp = kp * (kp > 0.2).astype(f32)
    cl = lax.broadcasted_iota(i32, (K, C), 1)
    kp = jnp.where(cl >= _NV, 0.0, kp)
    maxk = jnp.max(jnp.where(cl < C - 1, kp, 0.0), axis=1, keepdims=True)
    w = 1.0 - _BETA * u
    pk = jnp.where(u > maxk, 0.0, (obj * kp) * w)
    pu = (obj * u) * (1.0 - maxk)
    ent = jnp.where(cl == C - 1, pu, pk)
    ent = jnp.where(valid, ent, -1.0)

    ebits = lax.bitcast_convert_type(ent, i32)

    def body(_, lohi):
        lo, hi = lohi
        mid = lo + ((hi - lo + 1) >> 1)
        cnt = jnp.sum((ebits >= mid).astype(i32))
        ge = cnt >= _K_OUT
        return jnp.where(ge, mid, lo), jnp.where(ge, hi, mid - 1)

    lo2, _ = lax.fori_loop(
        0, 31, body, (jnp.int32(0), jnp.int32(0x7F800000)))

    q2 = (ebits >= lo2).astype(f32)
    rowsum = jnp.sum(q2, axis=1, keepdims=True)
    Lt = (ic < ir).astype(f32)
    ones_row = jnp.ones((1, K), f32)
    rowoff = _dot(Lt, rowsum)
    rowoffT = _dot(ones_row, eye * rowoff)
    rowendT = _dot(ones_row, eye * (rowoff + rowsum))
    sidx = lax.broadcasted_iota(i32, (K, K), 0).astype(f32)
    R2 = ((rowoffT <= sidx) & (sidx < rowendT)).astype(f32)
    RV = _dot(R2, ent)
    QR = _dot(R2, q2)
    SMg = _dot(R2, SM)
    s_local = lax.broadcasted_iota(i32, (K, 1), 0).astype(f32) - _dot(R2, rowoff)
    cumrow = QR
    sh = 1
    while sh < C:
        cumrow = cumrow + jnp.concatenate(
            [jnp.zeros((K, sh), f32), cumrow[:, : C - sh]], axis=1)
        sh *= 2
    C1 = ((cumrow == s_local + 1.0) & (QR > 0.5)).astype(f32)
    val = jnp.sum(C1 * RV, axis=1, keepdims=True)
    clf = lax.broadcasted_iota(i32, (K, C), 1).astype(f32)
    lab = jnp.sum(C1 * clf, axis=1, keepdims=True)

    valT = _dot(ones_row, eye * val)
    beats = (valT > val) | ((valT == val) & (ic < ir))
    rank = jnp.sum(beats.astype(f32), axis=1, keepdims=True)
    rankT = _dot(ones_row, eye * rank)
    F = (rankT == sidx).astype(f32)
    sc = _dot(F, val)
    lb = _dot(F, lab)
    bxg = _dot(F, SMg[:, 2:6])

    cx, cy, bw, bh = bxg[:, 0:1], bxg[:, 1:2], bxg[:, 2:3], bxg[:, 3:4]
    h_img = tsz_ref[0, :, 0:1]
    w_img = tsz_ref[0, :, 1:2]
    x0 = (cx - 0.5 * bw) * w_img
    y0 = (cy - 0.5 * bh) * h_img
    x1 = (cx + 0.5 * bw) * w_img
    y1 = (cy + 0.5 * bh) * h_img
    sc_ref[...] = sc[None]
    lb_ref[...] = lb.astype(i32)[None]
    bx_ref[...] = jnp.concatenate([x0, y0, x1, y1], axis=1)[None]


def _select(pred_logits, small, M, thr, tszf):
    B, N, C = pred_logits.shape
    return pl.pallas_call(
        _select_kernel,
        grid=(B,),
        in_specs=[
            pl.BlockSpec((1, N, C), lambda b: (b, 0, 0)),
            pl.BlockSpec((1, 8, N), lambda b: (b, 0, 0)),
            pl.BlockSpec((1, 1, N), lambda b: (b, 0, 0)),
            pl.BlockSpec((1, 8, 128), lambda b: (b, 0, 0)),
            pl.BlockSpec((1, 1, 2), lambda b: (b, 0, 0)),
        ],
        out_specs=[
            pl.BlockSpec((1, _K, 1), lambda b: (b, 0, 0)),
            pl.BlockSpec((1, _K, 1), lambda b: (b, 0, 0)),
            pl.BlockSpec((1, _K, 4), lambda b: (b, 0, 0)),
        ],
        out_shape=[
            jax.ShapeDtypeStruct((B, _K, 1), jnp.float32),
            jax.ShapeDtypeStruct((B, _K, 1), jnp.int32),
            jax.ShapeDtypeStruct((B, _K, 4), jnp.float32),
        ],
        interpret=_INTERPRET,
    )(pred_logits, small, M.reshape(B, 1, N), thr, tszf)


def kernel(pred_logits, pred_obj, pred_boxes, pred_unk, target_sizes):
    B, N, C = pred_logits.shape
    M = _row_max(pred_logits, pred_obj, pred_unk)
    thr = _thresholds(M)
    small = jnp.concatenate(
        [pred_obj[:, None, :], pred_unk[:, None, :],
         jnp.moveaxis(pred_boxes, 2, 1),
         jnp.zeros((B, 2, N), jnp.float32)], axis=1)
    tszf = target_sizes.astype(jnp.float32).reshape(B, 1, 2)
    sc, lb, bx = _select(pred_logits, small, M, thr, tszf)
    return (sc[:, :_K_OUT, 0], lb[:, :_K_OUT, 0], bx[:, :_K_OUT, :])
